# Initial kernel scaffold; baseline (speedup 1.0000x reference)
import jax
import jax.numpy as jnp
from jax.experimental import pallas as pl


def kernel(x, codebooks):
    raise NotImplementedError("write your pallas kernel here")


# speedup vs baseline: 1.1476x; 1.1476x over previous
import functools

import jax
import jax.numpy as jnp
from jax import lax
from jax.experimental import pallas as pl
from jax.experimental.pallas import tpu as pltpu

_M = 512
_KC = 2048


def _argmin_body(r_ref, rr_ref, cb_ref, cbn_ref, idx_ref, *, n_chunks, kc):
    m = r_ref.shape[0]
    k_total = n_chunks * kc
    rb = r_ref[...].astype(jnp.bfloat16)
    rr = rr_ref[...]

    run_min = jnp.full((m, 1), jnp.inf, jnp.float32)
    run_idx = jnp.zeros((m, 1), jnp.int32)
    for c in range(n_chunks):
        cbk = cb_ref[pl.ds(c * kc, kc), :].astype(jnp.bfloat16)
        e = lax.dot_general(rb, cbk, (((1,), (1,)), ((), ())),
                            preferred_element_type=jnp.float32)
        nk = cbn_ref[:, pl.ds(c * kc, kc)]
        sc = (rr - 2.0 * e) + nk
        cmin = jnp.min(sc, axis=1, keepdims=True)
        io = lax.broadcasted_iota(jnp.int32, (m, kc), 1) + c * kc
        carg = jnp.min(jnp.where(sc == cmin, io, k_total), axis=1, keepdims=True)
        upd = cmin < run_min
        run_min = jnp.where(upd, cmin, run_min)
        run_idx = jnp.where(upd, carg, run_idx)

    idx_ref[...] = run_idx


def _stage_indices(r, rr, cb, cbn):
    rows, d = r.shape
    k = cb.shape[0]
    idx = pl.pallas_call(
        functools.partial(_argmin_body, n_chunks=k // _KC, kc=_KC),
        grid=(rows // _M,),
        in_specs=[
            pl.BlockSpec((_M, d), lambda i: (i, 0)),
            pl.BlockSpec((_M, 1), lambda i: (i, 0)),
            pl.BlockSpec((k, d), lambda i: (0, 0)),
            pl.BlockSpec((1, k), lambda i: (0, 0)),
        ],
        out_specs=pl.BlockSpec((_M, 1), lambda i: (i, 0)),
        out_shape=jax.ShapeDtypeStruct((rows, 1), jnp.int32),
        compiler_params=pltpu.CompilerParams(
            dimension_semantics=("arbitrary",)),
    )(r, rr, cb, cbn)
    return idx[:, 0]


def kernel(x, codebooks):
    b, d, n = x.shape
    nq, k, _ = codebooks.shape
    rows = b * n

    r = jnp.transpose(x, (0, 2, 1))
    quant_out = jnp.zeros_like(r)
    for q in range(nq):
        cb = codebooks[q]
        rr = jnp.sum(r * r, axis=-1, keepdims=True)
        cbn = jnp.sum(cb * cb, axis=-1)
        idx = _stage_indices(r.reshape(rows, d), rr.reshape(rows, 1),
                             cb, cbn[None, :]).reshape(b, n)
        quant = jnp.take(cb, idx, axis=0)
        quant_out = quant_out + r + (quant - r)
        r = r - quant

    return jnp.transpose(quant_out, (0, 2, 1))


# speedup vs baseline: 1.3119x; 1.1431x over previous
import functools

import jax
import jax.numpy as jnp
from jax import lax
from jax.experimental import pallas as pl
from jax.experimental.pallas import tpu as pltpu
from jax.experimental.pallas import tpu_sc as plsc


def _make_sc_gather(k, d, rows):
    info = plsc.get_sparse_core_info()
    nw = info.num_cores * info.num_subcores
    b_per_w = rows // nw
    ch = 128
    mesh = plsc.VectorSubcoreMesh(core_axis_name="c", subcore_axis_name="s")

    @functools.partial(
        pl.kernel, mesh=mesh,
        out_type=jax.ShapeDtypeStruct((rows, d), jnp.float32),
        scratch_types=[
            pltpu.VMEM((ch,), jnp.int32),
            pltpu.VMEM((ch, d), jnp.float32),
            pltpu.SemaphoreType.DMA,
        ],
    )
    def gather(table_hbm, idx_hbm, out_hbm, idx_v, rows_v, sem):
        wid = lax.axis_index("s") * info.num_cores + lax.axis_index("c")
        base = wid * b_per_w
        for c in range(b_per_w // ch):
            off = base + c * ch
            pltpu.sync_copy(idx_hbm.at[pl.ds(off, ch)], idx_v)
            pltpu.async_copy(table_hbm.at[idx_v], rows_v, sem).wait()
            pltpu.sync_copy(rows_v, out_hbm.at[pl.ds(off, ch)])

    return gather

_M = 512
_KC = 2048


def _argmin_body(r_ref, rr_ref, cb_ref, cbn_ref, idx_ref, *, n_chunks, kc):
    m = r_ref.shape[0]
    k_total = n_chunks * kc
    rb = r_ref[...].astype(jnp.bfloat16)
    rr = rr_ref[...]

    run_min = jnp.full((m, 1), jnp.inf, jnp.float32)
    run_idx = jnp.zeros((m, 1), jnp.int32)
    for c in range(n_chunks):
        cbk = cb_ref[pl.ds(c * kc, kc), :].astype(jnp.bfloat16)
        e = lax.dot_general(rb, cbk, (((1,), (1,)), ((), ())),
                            preferred_element_type=jnp.float32)
        nk = cbn_ref[:, pl.ds(c * kc, kc)]
        sc = (rr - 2.0 * e) + nk
        cmin = jnp.min(sc, axis=1, keepdims=True)
        io = lax.broadcasted_iota(jnp.int32, (m, kc), 1) + c * kc
        carg = jnp.min(jnp.where(sc == cmin, io, k_total), axis=1, keepdims=True)
        upd = cmin < run_min
        run_min = jnp.where(upd, cmin, run_min)
        run_idx = jnp.where(upd, carg, run_idx)

    idx_ref[...] = run_idx


def _stage_indices(r, rr, cb, cbn):
    rows, d = r.shape
    k = cb.shape[0]
    idx = pl.pallas_call(
        functools.partial(_argmin_body, n_chunks=k // _KC, kc=_KC),
        grid=(rows // _M,),
        in_specs=[
            pl.BlockSpec((_M, d), lambda i: (i, 0)),
            pl.BlockSpec((_M, 1), lambda i: (i, 0)),
            pl.BlockSpec((k, d), lambda i: (0, 0)),
            pl.BlockSpec((1, k), lambda i: (0, 0)),
        ],
        out_specs=pl.BlockSpec((_M, 1), lambda i: (i, 0)),
        out_shape=jax.ShapeDtypeStruct((rows, 1), jnp.int32),
        compiler_params=pltpu.CompilerParams(
            dimension_semantics=("arbitrary",)),
    )(r, rr, cb, cbn)
    return idx[:, 0]


def kernel(x, codebooks):
    b, d, n = x.shape
    nq, k, _ = codebooks.shape
    rows = b * n

    sc_gather = _make_sc_gather(k, d, rows)

    r = jnp.transpose(x, (0, 2, 1))
    quant_out = jnp.zeros_like(r)
    for q in range(nq):
        cb = codebooks[q]
        rr = jnp.sum(r * r, axis=-1, keepdims=True)
        cbn = jnp.sum(cb * cb, axis=-1)
        idx = _stage_indices(r.reshape(rows, d), rr.reshape(rows, 1),
                             cb, cbn[None, :])
        quant = sc_gather(cb, idx).reshape(b, n, d)
        quant_out = quant_out + r + (quant - r)
        r = r - quant

    return jnp.transpose(quant_out, (0, 2, 1))
